# Initial kernel scaffold; baseline (speedup 1.0000x reference)
import jax
import jax.numpy as jnp
from jax.experimental import pallas as pl


def kernel(inputs, supports, W, b):
    raise NotImplementedError("write your pallas kernel here")


# speedup vs baseline: 2.1441x; 2.1441x over previous
import functools

import jax
import jax.numpy as jnp
from jax.experimental import pallas as pl
from jax.experimental.pallas import tpu as pltpu

_N_SUPPORTS = 2
_MAX_STEP = 2
_BN = 256


def _gc_body(n_batch, d_in, s_ref, x0_ref, w0_ref, wp_ref, b_ref, out_ref,
             xcur_ref):
    p = pl.program_id(0)
    nb = pl.program_id(1)
    rows = pl.ds(nb * _BN, _BN)

    s_blk = s_ref[0].astype(jnp.bfloat16)

    def _proj(xb, w_ref2, accumulate):
        for b in range(n_batch):
            contrib = jnp.dot(xb[:, b * d_in:(b + 1) * d_in], w_ref2[0],
                              preferred_element_type=jnp.float32)
            if accumulate:
                out_ref[b, rows, :] += contrib
            else:
                out_ref[b, rows, :] = contrib + b_ref[0, :][None, :]

    @pl.when(p == 0)
    def _init():
        _proj(x0_ref[rows, :], w0_ref, accumulate=False)

    @pl.when(p % 2 == 0)
    def _step1():
        y = jnp.dot(s_blk, x0_ref[...], preferred_element_type=jnp.float32)
        yb = y.astype(jnp.bfloat16)
        xcur_ref[rows, :] = yb
        _proj(yb, wp_ref, accumulate=True)

    @pl.when(p % 2 == 1)
    def _step2():
        y = jnp.dot(s_blk, xcur_ref[...], preferred_element_type=jnp.float32)
        x2 = 2.0 * y - x0_ref[rows, :].astype(jnp.float32)
        _proj(x2.astype(jnp.bfloat16), wp_ref, accumulate=True)


@jax.jit
def kernel(inputs, supports, W, b):
    B, N, D = inputs.shape
    OUT = W.shape[1]
    M = _N_SUPPORTS * _MAX_STEP + 1

    x0 = jnp.transpose(inputs, (1, 0, 2)).reshape(N, B * D)
    x0 = x0.astype(jnp.bfloat16)
    w_m = jnp.transpose(W.reshape(D, M, OUT), (1, 0, 2)).astype(jnp.bfloat16)
    b2 = b.reshape(1, OUT)

    n_phases = _N_SUPPORTS * _MAX_STEP
    grid = (n_phases, N // _BN)

    out = pl.pallas_call(
        functools.partial(_gc_body, B, D),
        grid=grid,
        in_specs=[
            pl.BlockSpec((1, _BN, N), lambda p, nb: (p // 2, nb, 0)),
            pl.BlockSpec((N, B * D), lambda p, nb: (0, 0)),
            pl.BlockSpec((1, D, OUT), lambda p, nb: (0, 0, 0)),
            pl.BlockSpec((1, D, OUT), lambda p, nb: (p + 1, 0, 0)),
            pl.BlockSpec((1, OUT), lambda p, nb: (0, 0)),
        ],
        out_specs=pl.BlockSpec((B, N, OUT), lambda p, nb: (0, 0, 0)),
        out_shape=jax.ShapeDtypeStruct((B, N, OUT), jnp.float32),
        scratch_shapes=[pltpu.VMEM((N, B * D), jnp.bfloat16)],
    )(supports, x0, w_m, w_m, b2)
    return out


# speedup vs baseline: 2.5414x; 1.1853x over previous
import functools

import jax
import jax.numpy as jnp
from jax.experimental import pallas as pl
from jax.experimental.pallas import tpu as pltpu

_N_SUPPORTS = 2
_MAX_STEP = 2
_BN = 512


def _gc_body(n_batch, d_in, s_ref, x0_ref, w0_ref, wp_ref, b_ref, out_ref,
             xcur_ref):
    p = pl.program_id(0)
    nb = pl.program_id(1)
    rows = pl.ds(nb * _BN, _BN)

    s_blk = s_ref[0].astype(jnp.bfloat16)

    def _proj(xb, w_ref2, accumulate):
        for b in range(n_batch):
            contrib = jnp.dot(xb[:, b * d_in:(b + 1) * d_in], w_ref2[0],
                              preferred_element_type=jnp.float32)
            if accumulate:
                out_ref[b, rows, :] += contrib
            else:
                out_ref[b, rows, :] = contrib + b_ref[0, :][None, :]

    @pl.when(p == 0)
    def _init():
        _proj(x0_ref[rows, :], w0_ref, accumulate=False)

    @pl.when(p % 2 == 0)
    def _step1():
        y = jnp.dot(s_blk, x0_ref[...], preferred_element_type=jnp.float32)
        yb = y.astype(jnp.bfloat16)
        xcur_ref[rows, :] = yb
        _proj(yb, wp_ref, accumulate=True)

    @pl.when(p % 2 == 1)
    def _step2():
        y = jnp.dot(s_blk, xcur_ref[...], preferred_element_type=jnp.float32)
        x2 = 2.0 * y - x0_ref[rows, :].astype(jnp.float32)
        _proj(x2.astype(jnp.bfloat16), wp_ref, accumulate=True)


@jax.jit
def kernel(inputs, supports, W, b):
    B, N, D = inputs.shape
    OUT = W.shape[1]
    M = _N_SUPPORTS * _MAX_STEP + 1

    x0 = jnp.transpose(inputs, (1, 0, 2)).reshape(N, B * D)
    x0 = x0.astype(jnp.bfloat16)
    w_m = jnp.transpose(W.reshape(D, M, OUT), (1, 0, 2)).astype(jnp.bfloat16)
    b2 = b.reshape(1, OUT)

    n_phases = _N_SUPPORTS * _MAX_STEP
    grid = (n_phases, N // _BN)

    out = pl.pallas_call(
        functools.partial(_gc_body, B, D),
        grid=grid,
        in_specs=[
            pl.BlockSpec((1, _BN, N), lambda p, nb: (p // 2, nb, 0)),
            pl.BlockSpec((N, B * D), lambda p, nb: (0, 0)),
            pl.BlockSpec((1, D, OUT), lambda p, nb: (0, 0, 0)),
            pl.BlockSpec((1, D, OUT), lambda p, nb: (p + 1, 0, 0)),
            pl.BlockSpec((1, OUT), lambda p, nb: (0, 0)),
        ],
        out_specs=pl.BlockSpec((B, N, OUT), lambda p, nb: (0, 0, 0)),
        out_shape=jax.ShapeDtypeStruct((B, N, OUT), jnp.float32),
        scratch_shapes=[pltpu.VMEM((N, B * D), jnp.bfloat16)],
    )(supports, x0, w_m, w_m, b2)
    return out


# speedup vs baseline: 2.7199x; 1.0702x over previous
import functools

import jax
import jax.numpy as jnp
from jax.experimental import pallas as pl
from jax.experimental.pallas import tpu as pltpu

_N_SUPPORTS = 2
_MAX_STEP = 2
_BN = 1024


def _gc_body(n_batch, d_in, s_ref, x0_ref, w0_ref, wp_ref, b_ref, out_ref,
             xcur_ref):
    p = pl.program_id(0)
    nb = pl.program_id(1)
    rows = pl.ds(nb * _BN, _BN)

    s_blk = s_ref[0].astype(jnp.bfloat16)

    def _proj(xb, w_ref2, accumulate):
        for b in range(n_batch):
            contrib = jnp.dot(xb[:, b * d_in:(b + 1) * d_in], w_ref2[0],
                              preferred_element_type=jnp.float32)
            if accumulate:
                out_ref[b, rows, :] += contrib
            else:
                out_ref[b, rows, :] = contrib + b_ref[0, :][None, :]

    @pl.when(p == 0)
    def _init():
        _proj(x0_ref[rows, :], w0_ref, accumulate=False)

    @pl.when(p % 2 == 0)
    def _step1():
        y = jnp.dot(s_blk, x0_ref[...], preferred_element_type=jnp.float32)
        yb = y.astype(jnp.bfloat16)
        xcur_ref[rows, :] = yb
        _proj(yb, wp_ref, accumulate=True)

    @pl.when(p % 2 == 1)
    def _step2():
        y = jnp.dot(s_blk, xcur_ref[...], preferred_element_type=jnp.float32)
        x2 = 2.0 * y - x0_ref[rows, :].astype(jnp.float32)
        _proj(x2.astype(jnp.bfloat16), wp_ref, accumulate=True)


@jax.jit
def kernel(inputs, supports, W, b):
    B, N, D = inputs.shape
    OUT = W.shape[1]
    M = _N_SUPPORTS * _MAX_STEP + 1

    x0 = jnp.transpose(inputs, (1, 0, 2)).reshape(N, B * D)
    x0 = x0.astype(jnp.bfloat16)
    w_m = jnp.transpose(W.reshape(D, M, OUT), (1, 0, 2)).astype(jnp.bfloat16)
    b2 = b.reshape(1, OUT)

    n_phases = _N_SUPPORTS * _MAX_STEP
    grid = (n_phases, N // _BN)

    out = pl.pallas_call(
        functools.partial(_gc_body, B, D),
        grid=grid,
        in_specs=[
            pl.BlockSpec((1, _BN, N), lambda p, nb: (p // 2, nb, 0)),
            pl.BlockSpec((N, B * D), lambda p, nb: (0, 0)),
            pl.BlockSpec((1, D, OUT), lambda p, nb: (0, 0, 0)),
            pl.BlockSpec((1, D, OUT), lambda p, nb: (p + 1, 0, 0)),
            pl.BlockSpec((1, OUT), lambda p, nb: (0, 0)),
        ],
        out_specs=pl.BlockSpec((B, N, OUT), lambda p, nb: (0, 0, 0)),
        out_shape=jax.ShapeDtypeStruct((B, N, OUT), jnp.float32),
        scratch_shapes=[pltpu.VMEM((N, B * D), jnp.bfloat16)],
    )(supports, x0, w_m, w_m, b2)
    return out
